# Initial kernel scaffold; baseline (speedup 1.0000x reference)
import jax
import jax.numpy as jnp
from jax.experimental import pallas as pl


def kernel(x, mapping_weight):
    raise NotImplementedError("write your pallas kernel here")


# speedup vs baseline: 1.5543x; 1.5543x over previous
import functools

import jax
import jax.numpy as jnp
from jax import lax
from jax.experimental import pallas as pl

N_ATTR = 100
N_VAL = 100
BATCH = 4096
B_BLK = 256


def _tc_body(x_ref, w_ref, msg_ref):
    w = w_ref[...].astype(jnp.float32)
    for a in range(N_ATTR):
        seg = x_ref[:, a * N_VAL:(a + 1) * N_VAL]
        mx = jnp.max(seg, axis=1, keepdims=True)
        iota = lax.broadcasted_iota(jnp.int32, seg.shape, 1)
        cand = jnp.where(seg == mx, iota, N_VAL)
        idxv = jnp.min(cand, axis=1, keepdims=True)
        oh = (iota == idxv).astype(jnp.float32)
        pair = jax.lax.dot(oh, w)
        msg_ref[:, 2 * a:2 * a + 2] = pair.astype(jnp.int32)


@jax.jit
def kernel(x, mapping_weight):
    batch = x.shape[0]
    grid = (batch // B_BLK,)
    msg = pl.pallas_call(
        _tc_body,
        grid=grid,
        in_specs=[
            pl.BlockSpec((B_BLK, N_ATTR * N_VAL), lambda i: (i, 0)),
            pl.BlockSpec((N_VAL, 2), lambda i: (0, 0)),
        ],
        out_specs=pl.BlockSpec((B_BLK, 2 * N_ATTR), lambda i: (i, 0)),
        out_shape=jax.ShapeDtypeStruct((batch, 2 * N_ATTR), jnp.int32),
    )(x, mapping_weight)
    zeros = jnp.zeros((batch, 2 * N_ATTR), dtype=jnp.float32)
    return (msg, zeros, zeros)


# speedup vs baseline: 2.0242x; 1.3023x over previous
import functools

import numpy as np
import jax
import jax.numpy as jnp
from jax import lax
from jax.experimental import pallas as pl
from jax.experimental.pallas import tpu as pltpu
from jax.experimental.pallas import tpu_sc as plsc

N_ATTR = 100
N_VAL = 100
ROW = N_ATTR * N_VAL
BATCH = 4096
OUT = 2 * N_ATTR
L = 16
NCHUNK = ROW // L
NTAB = (N_VAL + L - 1) // L
NWORKER = 32
ROWS_PER = BATCH // NWORKER
BIG = np.int32(2 ** 30)


def _perm(v, idx):
    return v.at[idx].get(mode="promise_in_bounds")


def _row_compute(rowbuf, w0buf, w1buf, outbuf):
    lane = lax.iota(jnp.int32, L)
    half = lane >> 1
    even = (lane & 1) == 0
    si = jnp.zeros((L,), jnp.int32)
    tab0 = [w0buf[pl.ds(L * c, L)] for c in range(NTAB)]
    tab1 = [w1buf[pl.ds(L * c, L)] for c in range(NTAB)]
    chunk_cache = {}

    def chunk(k):
        if k not in chunk_cache:
            chunk_cache[k] = (rowbuf[pl.ds(L * k, L)], lane + L * k)
        return chunk_cache[k]

    for s in range(N_ATTR):
        lo = N_VAL * s
        hi = lo + N_VAL
        k0 = lo // L
        k1 = (hi - 1) // L
        m = None
        for k in range(k0, k1 + 1):
            start = L * k
            v, avk = chunk(k)
            full = start >= lo and start + L <= hi
            maskc = None
            if not full:
                maskc = avk >= lo if start < lo else avk < hi
            if m is None:
                if full:
                    m, mi = v, avk
                else:
                    m = jnp.where(maskc, v, -jnp.inf)
                    mi = jnp.where(maskc, avk, BIG)
            else:
                pred = v > m
                if not full:
                    pred = jnp.logical_and(pred, maskc)
                m = jnp.where(pred, v, m)
                mi = jnp.where(pred, avk, mi)
        g = m
        for sh in (8, 4, 2, 1):
            g = jnp.maximum(g, _perm(g, lane ^ sh))
        cand = jnp.where(m == g, mi, BIG)
        for sh in (8, 4, 2, 1):
            cand = jnp.minimum(cand, _perm(cand, lane ^ sh))
        si = jnp.where(lane == (s % L), cand - lo, si)
        if s % L == L - 1 or s == N_ATTR - 1:
            t = s // L
            c_idx = si >> 4
            w_idx = si & (L - 1)
            r0 = _perm(tab0[0], w_idx)
            r1 = _perm(tab1[0], w_idx)
            for c in range(1, NTAB):
                hit = c_idx == c
                r0 = jnp.where(hit, _perm(tab0[c], w_idx), r0)
                r1 = jnp.where(hit, _perm(tab1[c], w_idx), r1)
            i0 = r0.astype(jnp.int32)
            i1 = r1.astype(jnp.int32)
            olo = jnp.where(even, _perm(i0, half), _perm(i1, half))
            outbuf[pl.ds(2 * L * t, L)] = olo
            if 2 * L * t + 2 * L <= 208:
                ohi = jnp.where(even, _perm(i0, half + 8), _perm(i1, half + 8))
                outbuf[pl.ds(2 * L * t + L, L)] = ohi


def _sc_body(x_hbm, w0_hbm, w1_hbm, msg_hbm, rowbuf, w0buf, w1buf, outbuf):
    wid = lax.axis_index("s") * 2 + lax.axis_index("c")
    base = wid * ROWS_PER
    pltpu.sync_copy(w0_hbm, w0buf.at[pl.ds(0, N_VAL)])
    pltpu.sync_copy(w1_hbm, w1buf.at[pl.ds(0, N_VAL)])

    def row_body(r, carry):
        pltpu.sync_copy(x_hbm.at[base + r], rowbuf)
        _row_compute(rowbuf, w0buf, w1buf, outbuf)
        pltpu.sync_copy(outbuf.at[pl.ds(0, OUT)], msg_hbm.at[base + r])
        return carry

    lax.fori_loop(0, ROWS_PER, row_body, 0)


_sc_kernel = functools.partial(
    pl.kernel,
    mesh=plsc.VectorSubcoreMesh(core_axis_name="c", subcore_axis_name="s"),
    out_type=jax.ShapeDtypeStruct((BATCH, OUT), jnp.int32),
    compiler_params=pltpu.CompilerParams(use_tc_tiling_on_sc=False),
    scratch_types=[
        pltpu.VMEM((ROW,), jnp.float32),
        pltpu.VMEM((NTAB * L,), jnp.float32),
        pltpu.VMEM((NTAB * L,), jnp.float32),
        pltpu.VMEM((208,), jnp.int32),
    ],
)(_sc_body)


@jax.jit
def kernel(x, mapping_weight):
    w0 = mapping_weight[:, 0]
    w1 = mapping_weight[:, 1]
    msg = _sc_kernel(x, w0, w1)
    zeros = jnp.zeros((BATCH, OUT), dtype=jnp.float32)
    return (msg, zeros, zeros)


# speedup vs baseline: 2.1457x; 1.0600x over previous
import functools

import numpy as np
import jax
import jax.numpy as jnp
from jax import lax
from jax.experimental import pallas as pl
from jax.experimental.pallas import tpu as pltpu
from jax.experimental.pallas import tpu_sc as plsc

N_ATTR = 100
N_VAL = 100
ROW = N_ATTR * N_VAL
HALF = ROW // 2
NSEG = N_ATTR // 2
HOUT = NSEG * 2
HCPY = 104
BATCH = 4096
OUT = 2 * N_ATTR
OUTP = 224
L = 16
NTAB = (N_VAL + L - 1) // L
NWORKER = 32
ROWS_PER = BATCH // NWORKER
HPAD = 5008
OPAD = 112
BIGF = np.float32(2.0 ** 30)


def _perm(v, idx):
    return v.at[idx].get(mode="promise_in_bounds")


def _half_compute(buf, tab0, tab1, outbuf):
    lane = lax.iota(jnp.int32, L)
    lanef = lane.astype(jnp.float32)
    half = lane >> 1
    even = (lane & 1) == 0
    si = jnp.zeros((L,), jnp.float32)
    chunk_cache = {}

    def chunk(k):
        if k not in chunk_cache:
            chunk_cache[k] = (buf[pl.ds(L * k, L)], lanef + float(L * k))
        return chunk_cache[k]

    for s in range(NSEG):
        lo = N_VAL * s
        hi = lo + N_VAL
        k0 = lo // L
        k1 = (hi - 1) // L
        m = None
        for k in range(k0, k1 + 1):
            start = L * k
            v, avk = chunk(k)
            full = start >= lo and start + L <= hi
            maskc = None
            if not full:
                maskc = avk >= lo if start < lo else avk < hi
            if m is None:
                if full:
                    m, mi = v, avk
                else:
                    m = jnp.where(maskc, v, -jnp.inf)
                    mi = jnp.where(maskc, avk, BIGF)
            else:
                pred = v > m
                if not full:
                    pred = jnp.logical_and(pred, maskc)
                m = jnp.where(pred, v, m)
                mi = jnp.where(pred, avk, mi)
        g = m
        for sh in (8, 4, 2, 1):
            g = jnp.maximum(g, _perm(g, lane ^ sh))
        cand = jnp.where(m == g, mi, BIGF)
        for sh in (8, 4, 2, 1):
            cand = jnp.minimum(cand, _perm(cand, lane ^ sh))
        si = jnp.where(lane == (s % L), cand - float(lo), si)
        if s % L == L - 1 or s == NSEG - 1:
            t = s // L
            sii = si.astype(jnp.int32)
            c_idx = sii >> 4
            w_idx = sii & (L - 1)
            r0 = _perm(tab0[0], w_idx)
            r1 = _perm(tab1[0], w_idx)
            for c in range(1, NTAB):
                hit = c_idx == c
                r0 = jnp.where(hit, _perm(tab0[c], w_idx), r0)
                r1 = jnp.where(hit, _perm(tab1[c], w_idx), r1)
            i0 = r0.astype(jnp.int32)
            i1 = r1.astype(jnp.int32)
            olo = jnp.where(even, _perm(i0, half), _perm(i1, half))
            outbuf[pl.ds(2 * L * t, L)] = olo
            if 2 * L * t + 2 * L <= OPAD:
                ohi = jnp.where(even, _perm(i0, half + 8), _perm(i1, half + 8))
                outbuf[pl.ds(2 * L * t + L, L)] = ohi


def _sc_body(x_hbm, w0_hbm, w1_hbm, msg_hbm,
             bufA, bufB, w0buf, w1buf, outA, outB,
             semA, semB, osemA, osemB):
    wid = lax.axis_index("s") * 2 + lax.axis_index("c")
    base = wid * ROWS_PER
    last = base + ROWS_PER - 1
    pltpu.sync_copy(w0_hbm, w0buf.at[pl.ds(0, N_VAL)])
    pltpu.sync_copy(w1_hbm, w1buf.at[pl.ds(0, N_VAL)])
    tab0 = [w0buf[pl.ds(L * c, L)] for c in range(NTAB)]
    tab1 = [w1buf[pl.ds(L * c, L)] for c in range(NTAB)]

    dstA = bufA.at[pl.ds(0, HALF)]
    dstB = bufB.at[pl.ds(0, HALF)]

    def srcA(r):
        return x_hbm.at[r, pl.ds(0, HALF)]

    def srcB(r):
        return x_hbm.at[r, pl.ds(HALF, HALF)]

    pltpu.async_copy(srcA(base), dstA, semA)

    def row_body(i, carry):
        r = base + i
        pltpu.async_copy(srcB(r), dstB, semB)

        pltpu.make_async_copy(srcA(r), dstA, semA).wait()
        pl.when(i > 0)(
            lambda: pltpu.make_async_copy(
                outA.at[pl.ds(0, HCPY)], msg_hbm.at[r, pl.ds(0, HCPY)],
                osemA).wait())
        _half_compute(bufA, tab0, tab1, outA)
        pltpu.async_copy(
            outA.at[pl.ds(0, HCPY)], msg_hbm.at[r, pl.ds(0, HCPY)], osemA)

        pltpu.async_copy(srcA(jnp.minimum(r + 1, last)), dstA, semA)

        pltpu.make_async_copy(srcB(r), dstB, semB).wait()
        pl.when(i > 0)(
            lambda: pltpu.make_async_copy(
                outB.at[pl.ds(0, HCPY)], msg_hbm.at[r, pl.ds(OPAD, HCPY)],
                osemB).wait())
        _half_compute(bufB, tab0, tab1, outB)
        pltpu.async_copy(
            outB.at[pl.ds(0, HCPY)], msg_hbm.at[r, pl.ds(OPAD, HCPY)], osemB)
        return carry

    lax.fori_loop(0, ROWS_PER, row_body, 0)
    pltpu.make_async_copy(srcA(last), dstA, semA).wait()
    pltpu.make_async_copy(
        outA.at[pl.ds(0, HCPY)], msg_hbm.at[last, pl.ds(0, HCPY)], osemA).wait()
    pltpu.make_async_copy(
        outB.at[pl.ds(0, HCPY)], msg_hbm.at[last, pl.ds(OPAD, HCPY)], osemB).wait()


_sc_kernel = functools.partial(
    pl.kernel,
    mesh=plsc.VectorSubcoreMesh(core_axis_name="c", subcore_axis_name="s"),
    out_type=jax.ShapeDtypeStruct((BATCH, OUTP), jnp.int32),
    compiler_params=pltpu.CompilerParams(use_tc_tiling_on_sc=False),
    scratch_types=[
        pltpu.VMEM((HPAD,), jnp.float32),
        pltpu.VMEM((HPAD,), jnp.float32),
        pltpu.VMEM((NTAB * L,), jnp.float32),
        pltpu.VMEM((NTAB * L,), jnp.float32),
        pltpu.VMEM((OPAD,), jnp.int32),
        pltpu.VMEM((OPAD,), jnp.int32),
        pltpu.SemaphoreType.DMA,
        pltpu.SemaphoreType.DMA,
        pltpu.SemaphoreType.DMA,
        pltpu.SemaphoreType.DMA,
    ],
)(_sc_body)


@jax.jit
def kernel(x, mapping_weight):
    w0 = mapping_weight[:, 0]
    w1 = mapping_weight[:, 1]
    msgp = _sc_kernel(x, w0, w1)
    msg = jnp.concatenate(
        [msgp[:, :HOUT], msgp[:, OPAD:OPAD + HOUT]], axis=1)
    zeros = jnp.zeros((BATCH, OUT), dtype=jnp.float32)
    return (msg, zeros, zeros)
